# baseline (speedup 1.0000x reference)
import numpy as np

import jax
import jax.numpy as jnp
from jax.experimental import pallas as pl
from jax.experimental.pallas import tpu as pltpu

_B, _T, _D = 1, 2048, 1024
_H, _KV, _HD = 16, 4, 64
_COMP = 256
_SW = 512
_SBS = 128
_SNB = 8
_EPS = 1e-6
_SCALE = _HD ** -0.5
_NBLK = _T // _SBS
_G = _H // _KV
_KD = _KV * _HD

_TS = 256
_RBS = 256
_NP = 3
_NI = _T // _RBS
_QR = _G * _RBS
_NSUB = _TS // _SBS
_NI_BOOST = 3
_NEG = -1e30


def _proj_kernel(x_ref, wc_ref, bc_ref, wdk_ref, bdk_ref, wdv_ref, bdv_ref,
                 wq_ref, bq_ref, wkn_ref, wqn_ref, cos_ref, sin_ref,
                 sk_ref, sq_ref, sm_ref,
                 q_ref, k_ref, v_ref, km_ref):
    i = pl.program_id(0)
    x = x_ref[...]
    lat = jnp.dot(x, wc_ref[...], preferred_element_type=jnp.float32) + bc_ref[...]
    ka = jnp.dot(lat, wdk_ref[...], preferred_element_type=jnp.float32) + bdk_ref[...]
    v = jnp.dot(lat, wdv_ref[...], preferred_element_type=jnp.float32) + bdv_ref[...]
    qa = jnp.dot(x, wq_ref[...], preferred_element_type=jnp.float32) + bq_ref[...]
    cos = cos_ref[...]
    sin = sin_ref[...]

    def norm_rope_full(a, s_ref, w, n):
        ss = jnp.dot(a * a, s_ref[...], preferred_element_type=jnp.float32)
        inv = jax.lax.rsqrt(ss * (1.0 / _HD) + _EPS)
        t = a * w
        tsw = jnp.concatenate(
            [t[:, g * 32 + 32:g * 32 + 64] if g % 2 == 0
             else t[:, g * 32 - 32:g * 32] for g in range(2 * n)], axis=1)
        ct = jnp.concatenate([cos] * n, axis=1)
        st = jnp.concatenate([sin] * n, axis=1)
        return (t * ct + tsw * st) * inv

    k = norm_rope_full(ka, sk_ref, wkn_ref[...], _KV)
    q = norm_rope_full(qa, sq_ref, wqn_ref[...], _H)
    means = jnp.dot(sm_ref[...], k, preferred_element_type=jnp.float32)
    for h in range(_KV):
        k_ref[h] = k[:, h * _HD:(h + 1) * _HD]
        v_ref[h] = v[:, h * _HD:(h + 1) * _HD]
        km_ref[h, pl.ds(i * _NSUB, _NSUB), :] = means[:, h * _HD:(h + 1) * _HD]
    for h in range(_H):
        q_ref[h] = q[:, h * _HD:(h + 1) * _HD]


def _topk_boost_t(bst):
    rowid = jax.lax.broadcasted_iota(jnp.int32, bst.shape, 0)
    sel = []
    for c in range(_NBLK):
        bc = bst[c:c + 1, :]
        gt = jnp.sum((bst > bc).astype(jnp.float32), axis=0, keepdims=True)
        eq = jnp.sum(((bst == bc) & (rowid < c)).astype(jnp.float32),
                     axis=0, keepdims=True)
        sel.append((gt + eq < float(_SNB)).astype(jnp.float32))
    selt = jnp.concatenate(sel, axis=0)
    return jnp.transpose(bst * selt)


def _make_attn_kernel():
    def body(q_ref, k0, k1, k2, v0, v1, v2, m0, m1, m2, km_ref, wo_ref,
             bo_ref, out_ref, boost_ref):
        i = pl.program_id(0)
        g = pl.program_id(1)
        qs = q_ref[...].reshape(_QR, _HD)
        sparts = []
        for kp, mp in zip((k0, k1, k2), (m0, m1, m2)):
            sp = jax.lax.dot_general(qs, kp[0], (((1,), (1,)), ((), ())),
                                     preferred_element_type=jnp.float32)
            sparts.append(sp * _SCALE + mp[0])

        @pl.when(i < _NI_BOOST)
        def _boost():
            bst = jax.lax.dot_general(km_ref[0], qs, (((1,), (1,)), ((), ())),
                                      preferred_element_type=jnp.float32)
            boost_ref[...] = _topk_boost_t(bst * _SCALE)

        @pl.when((i == _NI_BOOST) & (g == 0))
        def _clear():
            boost_ref[...] = jnp.zeros((_QR, _NBLK), jnp.float32)

        s0 = sparts[0]
        sparts[0] = jnp.concatenate(
            [s0[:, :_NBLK] + boost_ref[...], s0[:, _NBLK:]], axis=1)

        m = jnp.maximum(
            jnp.maximum(jnp.max(sparts[0], axis=1, keepdims=True),
                        jnp.max(sparts[1], axis=1, keepdims=True)),
            jnp.max(sparts[2], axis=1, keepdims=True))
        o = None
        l = None
        for sp, vp in zip(sparts, (v0, v1, v2)):
            pp = jnp.exp(sp - m)
            ls = jnp.sum(pp, axis=1, keepdims=True)
            op = jnp.dot(pp, vp[0], preferred_element_type=jnp.float32)
            o = op if o is None else o + op
            l = ls if l is None else l + ls
        o = o * (1.0 / l)
        o_cat = jnp.concatenate(
            [o[hl * _RBS:(hl + 1) * _RBS] for hl in range(_G)], axis=1)
        proj = jnp.dot(o_cat, wo_ref[...], preferred_element_type=jnp.float32)

        @pl.when(g == 0)
        def _init():
            out_ref[...] = proj + bo_ref[...]

        @pl.when(g > 0)
        def _acc():
            out_ref[...] += proj

    return body


def _band_masks():
    r = np.arange(_QR)[:, None] % _RBS
    c = np.arange(_RBS)[None, :]

    def madd(abs_i, colblock):
        rows = abs_i * _RBS + r
        cols = colblock * _RBS + c
        ok = (cols <= rows) & (cols >= rows - _SW)
        return np.where(ok, 0.0, _NEG).astype(np.float32)

    return [jnp.asarray(np.stack(
        [madd(i, max(i - 2, 0) + p) for i in range(_NI_BOOST + 1)]))
        for p in range(_NP)]


def kernel(x, W_comp, b_comp, W_dk, b_dk, W_dv, b_dv, W_q, b_q, W_o, b_o,
           q_norm_w, k_norm_w):
    x2 = x[0]

    perm_h = np.concatenate([np.arange(0, _HD, 2), np.arange(1, _HD, 2)])
    perm_q = (np.arange(_H)[:, None] * _HD + perm_h[None, :]).reshape(-1)
    perm_k = (np.arange(_KV)[:, None] * _HD + perm_h[None, :]).reshape(-1)
    W_qA = W_q[:, perm_q]
    b_qA = b_q[perm_q][None, :]
    W_dkA = W_dk[:, perm_k]
    b_dkA = b_dk[perm_k][None, :]
    wqn = jnp.tile(q_norm_w[perm_h], (_H,))[None, :]
    wkn = jnp.tile(k_norm_w[perm_h], (_KV,))[None, :]

    pos = np.arange(_T, dtype=np.float32)
    inv_freq = 1.0 / (10000.0 ** (np.arange(0, _HD, 2, dtype=np.float32) / _HD))
    ang = pos[:, None] * inv_freq[None, :]
    cos64 = jnp.asarray(np.concatenate([np.cos(ang), np.cos(ang)], axis=1))
    sin64 = jnp.asarray(np.concatenate([-np.sin(ang), np.sin(ang)], axis=1))

    gid_k = np.arange(_KD) // _HD
    sk = jnp.asarray((gid_k[:, None] == gid_k[None, :]).astype(np.float32))
    gid_q = np.arange(_D) // _HD
    sq = jnp.asarray((gid_q[:, None] == gid_q[None, :]).astype(np.float32))
    sub = np.arange(_TS) // _SBS
    sm = jnp.asarray((np.arange(_NSUB)[:, None] == sub[None, :])
                     .astype(np.float32) / _SBS)

    q, k, v, km = pl.pallas_call(
        _proj_kernel,
        grid=(_T // _TS,),
        in_specs=[
            pl.BlockSpec((_TS, _D), lambda i: (i, 0)),
            pl.BlockSpec((_D, _COMP), lambda i: (0, 0)),
            pl.BlockSpec((1, _COMP), lambda i: (0, 0)),
            pl.BlockSpec((_COMP, _KD), lambda i: (0, 0)),
            pl.BlockSpec((1, _KD), lambda i: (0, 0)),
            pl.BlockSpec((_COMP, _KD), lambda i: (0, 0)),
            pl.BlockSpec((1, _KD), lambda i: (0, 0)),
            pl.BlockSpec((_D, _D), lambda i: (0, 0)),
            pl.BlockSpec((1, _D), lambda i: (0, 0)),
            pl.BlockSpec((1, _KD), lambda i: (0, 0)),
            pl.BlockSpec((1, _D), lambda i: (0, 0)),
            pl.BlockSpec((_TS, _HD), lambda i: (i, 0)),
            pl.BlockSpec((_TS, _HD), lambda i: (i, 0)),
            pl.BlockSpec((_KD, _KD), lambda i: (0, 0)),
            pl.BlockSpec((_D, _D), lambda i: (0, 0)),
            pl.BlockSpec((_NSUB, _TS), lambda i: (0, 0)),
        ],
        out_specs=[
            pl.BlockSpec((_H, _TS, _HD), lambda i: (0, i, 0)),
            pl.BlockSpec((_KV, _TS, _HD), lambda i: (0, i, 0)),
            pl.BlockSpec((_KV, _TS, _HD), lambda i: (0, i, 0)),
            pl.BlockSpec((_KV, _NBLK, _HD), lambda i: (0, 0, 0)),
        ],
        out_shape=[
            jax.ShapeDtypeStruct((_H, _T, _HD), jnp.float32),
            jax.ShapeDtypeStruct((_KV, _T, _HD), jnp.float32),
            jax.ShapeDtypeStruct((_KV, _T, _HD), jnp.float32),
            jax.ShapeDtypeStruct((_KV, _NBLK, _HD), jnp.float32),
        ],
        compiler_params=pltpu.CompilerParams(
            dimension_semantics=("arbitrary",)),
    )(x2, W_comp, b_comp.reshape(1, -1), W_dkA, b_dkA,
      W_dv, b_dv.reshape(1, -1), W_qA, b_qA, wkn, wqn, cos64, sin64,
      sk, sq, sm)

    bo2 = b_o.reshape(1, -1)
    masks = _band_masks()

    def kv_part(p):
        return pl.BlockSpec(
            (1, _RBS, _HD),
            lambda i, g, p=p: (g, jnp.maximum(i - 2, 0) + p, 0))

    def mask_part(p):
        return pl.BlockSpec(
            (1, _QR, _RBS),
            lambda i, g, p=p: (jnp.minimum(i, _NI_BOOST), 0, 0))

    out = pl.pallas_call(
        _make_attn_kernel(),
        grid=(_NI, _KV),
        in_specs=[
            pl.BlockSpec((_G, _RBS, _HD), lambda i, g: (g, i, 0)),
            kv_part(0), kv_part(1), kv_part(2),
            kv_part(0), kv_part(1), kv_part(2),
            mask_part(0), mask_part(1), mask_part(2),
            pl.BlockSpec((1, _NBLK, _HD), lambda i, g: (g, 0, 0)),
            pl.BlockSpec((_G * _HD, _D), lambda i, g: (g, 0)),
            pl.BlockSpec((1, _D), lambda i, g: (0, 0)),
        ],
        out_specs=pl.BlockSpec((_RBS, _D), lambda i, g: (i, 0)),
        out_shape=jax.ShapeDtypeStruct((_T, _D), jnp.float32),
        scratch_shapes=[pltpu.VMEM((_QR, _NBLK), jnp.float32)],
        compiler_params=pltpu.CompilerParams(
            dimension_semantics=("parallel", "arbitrary")),
    )(q, k, k, k, v, v, v, masks[0], masks[1], masks[2], km, W_o, bo2)

    return out.reshape(_B, _T, _D)


# speedup vs baseline: 1.0191x; 1.0191x over previous
import numpy as np

import jax
import jax.numpy as jnp
from jax.experimental import pallas as pl
from jax.experimental.pallas import tpu as pltpu

_B, _T, _D = 1, 2048, 1024
_H, _KV, _HD = 16, 4, 64
_COMP = 256
_SW = 512
_SBS = 128
_SNB = 8
_EPS = 1e-6
_SCALE = _HD ** -0.5
_NBLK = _T // _SBS
_G = _H // _KV
_KD = _KV * _HD

_TS = 256
_RBS = 256
_NP = 3
_NI = _T // _RBS
_QR = _G * _RBS
_NSUB = _TS // _SBS
_NI_BOOST = 3
_NEG = -1e30


def _proj_kernel(x_ref, wc_ref, bc_ref, wdk_ref, bdk_ref, wdv_ref, bdv_ref,
                 wq_ref, bq_ref, wkn_ref, wqn_ref, cos_ref, sin_ref,
                 sk_ref, sq_ref, sm_ref,
                 q_ref, k_ref, v_ref, km_ref):
    i = pl.program_id(0)
    x = x_ref[...]
    lat = jnp.dot(x, wc_ref[...], preferred_element_type=jnp.float32) + bc_ref[...]
    ka = jnp.dot(lat, wdk_ref[...], preferred_element_type=jnp.float32) + bdk_ref[...]
    v = jnp.dot(lat, wdv_ref[...], preferred_element_type=jnp.float32) + bdv_ref[...]
    qa = jnp.dot(x, wq_ref[...], preferred_element_type=jnp.float32) + bq_ref[...]
    cos = cos_ref[...]
    sin = sin_ref[...]

    def norm_rope_full(a, s_ref, w, n):
        ss = jnp.dot(a * a, s_ref[...], preferred_element_type=jnp.float32)
        inv = jax.lax.rsqrt(ss * (1.0 / _HD) + _EPS)
        t = a * w
        tsw = jnp.concatenate(
            [t[:, g * 32 + 32:g * 32 + 64] if g % 2 == 0
             else t[:, g * 32 - 32:g * 32] for g in range(2 * n)], axis=1)
        ct = jnp.concatenate([cos] * n, axis=1)
        st = jnp.concatenate([sin] * n, axis=1)
        return (t * ct + tsw * st) * inv

    k = norm_rope_full(ka, sk_ref, wkn_ref[...], _KV)
    q = norm_rope_full(qa, sq_ref, wqn_ref[...], _H)
    means = jnp.dot(sm_ref[...], k, preferred_element_type=jnp.float32)
    for h in range(_KV):
        k_ref[h] = k[:, h * _HD:(h + 1) * _HD]
        v_ref[h] = v[:, h * _HD:(h + 1) * _HD]
        km_ref[h, pl.ds(i * _NSUB, _NSUB), :] = means[:, h * _HD:(h + 1) * _HD]
    for h in range(_H):
        q_ref[h] = q[:, h * _HD:(h + 1) * _HD]


def _topk_boost_t(bst):
    rowid = jax.lax.broadcasted_iota(jnp.int32, bst.shape, 0)
    sel = []
    for c in range(_NBLK):
        bc = bst[c:c + 1, :]
        gt = jnp.sum((bst > bc).astype(jnp.float32), axis=0, keepdims=True)
        eq = jnp.sum(((bst == bc) & (rowid < c)).astype(jnp.float32),
                     axis=0, keepdims=True)
        sel.append((gt + eq < float(_SNB)).astype(jnp.float32))
    selt = jnp.concatenate(sel, axis=0)
    return jnp.transpose(bst * selt)


def _make_attn_kernel(with_boost):
    def body(*refs):
        if with_boost:
            (q_ref, k0, k1, k2, v0, v1, v2, m0, m1, m2, km_ref, wo_ref,
             bo_ref, out_ref) = refs
        else:
            (q_ref, k0, k1, k2, v0, v1, v2, m0, m1, m2, wo_ref,
             bo_ref, out_ref) = refs
        g = pl.program_id(1)
        qs = q_ref[...].reshape(_QR, _HD)
        sparts = []
        for kp, mp in zip((k0, k1, k2), (m0, m1, m2)):
            sp = jax.lax.dot_general(qs, kp[0], (((1,), (1,)), ((), ())),
                                     preferred_element_type=jnp.float32)
            sparts.append(sp * _SCALE + mp[0])

        if with_boost:
            bst = jax.lax.dot_general(km_ref[0], qs, (((1,), (1,)), ((), ())),
                                      preferred_element_type=jnp.float32)
            boost = _topk_boost_t(bst * _SCALE)
            s0 = sparts[0]
            sparts[0] = jnp.concatenate(
                [s0[:, :_NBLK] + boost, s0[:, _NBLK:]], axis=1)

        m = jnp.maximum(
            jnp.maximum(jnp.max(sparts[0], axis=1, keepdims=True),
                        jnp.max(sparts[1], axis=1, keepdims=True)),
            jnp.max(sparts[2], axis=1, keepdims=True))
        o = None
        l = None
        for sp, vp in zip(sparts, (v0, v1, v2)):
            pp = jnp.exp(sp - m)
            ls = jnp.sum(pp, axis=1, keepdims=True)
            op = jnp.dot(pp, vp[0], preferred_element_type=jnp.float32)
            o = op if o is None else o + op
            l = ls if l is None else l + ls
        o = o * (1.0 / l)
        o_cat = jnp.concatenate(
            [o[hl * _RBS:(hl + 1) * _RBS] for hl in range(_G)], axis=1)
        proj = jnp.dot(o_cat, wo_ref[...], preferred_element_type=jnp.float32)

        @pl.when(g == 0)
        def _init():
            out_ref[...] = proj + bo_ref[...]

        @pl.when(g > 0)
        def _acc():
            out_ref[...] += proj

    return body


def _band_masks():
    r = np.arange(_QR)[:, None] % _RBS
    c = np.arange(_RBS)[None, :]

    def madd(abs_i, colblock):
        rows = abs_i * _RBS + r
        cols = colblock * _RBS + c
        ok = (cols <= rows) & (cols >= rows - _SW)
        return np.where(ok, 0.0, _NEG).astype(np.float32)

    m1 = [np.stack([madd(i, p) for i in range(_NI_BOOST)])
          for p in range(_NP)]
    m2 = [madd(3, 1 + p)[None] for p in range(_NP)]
    return [jnp.asarray(m) for m in m1], [jnp.asarray(m) for m in m2]


def kernel(x, W_comp, b_comp, W_dk, b_dk, W_dv, b_dv, W_q, b_q, W_o, b_o,
           q_norm_w, k_norm_w):
    x2 = x[0]

    perm_h = np.concatenate([np.arange(0, _HD, 2), np.arange(1, _HD, 2)])
    perm_q = (np.arange(_H)[:, None] * _HD + perm_h[None, :]).reshape(-1)
    perm_k = (np.arange(_KV)[:, None] * _HD + perm_h[None, :]).reshape(-1)
    W_qA = W_q[:, perm_q]
    b_qA = b_q[perm_q][None, :]
    W_dkA = W_dk[:, perm_k]
    b_dkA = b_dk[perm_k][None, :]
    wqn = jnp.tile(q_norm_w[perm_h], (_H,))[None, :]
    wkn = jnp.tile(k_norm_w[perm_h], (_KV,))[None, :]

    pos = np.arange(_T, dtype=np.float32)
    inv_freq = 1.0 / (10000.0 ** (np.arange(0, _HD, 2, dtype=np.float32) / _HD))
    ang = pos[:, None] * inv_freq[None, :]
    cos64 = jnp.asarray(np.concatenate([np.cos(ang), np.cos(ang)], axis=1))
    sin64 = jnp.asarray(np.concatenate([-np.sin(ang), np.sin(ang)], axis=1))

    gid_k = np.arange(_KD) // _HD
    sk = jnp.asarray((gid_k[:, None] == gid_k[None, :]).astype(np.float32))
    gid_q = np.arange(_D) // _HD
    sq = jnp.asarray((gid_q[:, None] == gid_q[None, :]).astype(np.float32))
    sub = np.arange(_TS) // _SBS
    sm = jnp.asarray((np.arange(_NSUB)[:, None] == sub[None, :])
                     .astype(np.float32) / _SBS)

    q, k, v, km = pl.pallas_call(
        _proj_kernel,
        grid=(_T // _TS,),
        in_specs=[
            pl.BlockSpec((_TS, _D), lambda i: (i, 0)),
            pl.BlockSpec((_D, _COMP), lambda i: (0, 0)),
            pl.BlockSpec((1, _COMP), lambda i: (0, 0)),
            pl.BlockSpec((_COMP, _KD), lambda i: (0, 0)),
            pl.BlockSpec((1, _KD), lambda i: (0, 0)),
            pl.BlockSpec((_COMP, _KD), lambda i: (0, 0)),
            pl.BlockSpec((1, _KD), lambda i: (0, 0)),
            pl.BlockSpec((_D, _D), lambda i: (0, 0)),
            pl.BlockSpec((1, _D), lambda i: (0, 0)),
            pl.BlockSpec((1, _KD), lambda i: (0, 0)),
            pl.BlockSpec((1, _D), lambda i: (0, 0)),
            pl.BlockSpec((_TS, _HD), lambda i: (i, 0)),
            pl.BlockSpec((_TS, _HD), lambda i: (i, 0)),
            pl.BlockSpec((_KD, _KD), lambda i: (0, 0)),
            pl.BlockSpec((_D, _D), lambda i: (0, 0)),
            pl.BlockSpec((_NSUB, _TS), lambda i: (0, 0)),
        ],
        out_specs=[
            pl.BlockSpec((_H, _TS, _HD), lambda i: (0, i, 0)),
            pl.BlockSpec((_KV, _TS, _HD), lambda i: (0, i, 0)),
            pl.BlockSpec((_KV, _TS, _HD), lambda i: (0, i, 0)),
            pl.BlockSpec((_KV, _NBLK, _HD), lambda i: (0, 0, 0)),
        ],
        out_shape=[
            jax.ShapeDtypeStruct((_H, _T, _HD), jnp.float32),
            jax.ShapeDtypeStruct((_KV, _T, _HD), jnp.float32),
            jax.ShapeDtypeStruct((_KV, _T, _HD), jnp.float32),
            jax.ShapeDtypeStruct((_KV, _NBLK, _HD), jnp.float32),
        ],
        compiler_params=pltpu.CompilerParams(
            dimension_semantics=("arbitrary",)),
    )(x2, W_comp, b_comp.reshape(1, -1), W_dkA, b_dkA,
      W_dv, b_dv.reshape(1, -1), W_qA, b_qA, wkn, wqn, cos64, sin64,
      sk, sq, sm)

    bo2 = b_o.reshape(1, -1)
    masks1, masks2 = _band_masks()

    def kv_part(p, shift):
        if shift is None:
            return pl.BlockSpec((1, _RBS, _HD), lambda i, g, p=p: (g, p, 0))
        return pl.BlockSpec((1, _RBS, _HD),
                            lambda i, g, p=p, s=shift: (g, i + s + p, 0))

    out1 = pl.pallas_call(
        _make_attn_kernel(True),
        grid=(_NI_BOOST, _KV),
        in_specs=[
            pl.BlockSpec((_G, _RBS, _HD), lambda i, g: (g, i, 0)),
            kv_part(0, None), kv_part(1, None), kv_part(2, None),
            kv_part(0, None), kv_part(1, None), kv_part(2, None),
            pl.BlockSpec((1, _QR, _RBS), lambda i, g: (i, 0, 0)),
            pl.BlockSpec((1, _QR, _RBS), lambda i, g: (i, 0, 0)),
            pl.BlockSpec((1, _QR, _RBS), lambda i, g: (i, 0, 0)),
            pl.BlockSpec((1, _NBLK, _HD), lambda i, g: (g, 0, 0)),
            pl.BlockSpec((_G * _HD, _D), lambda i, g: (g, 0)),
            pl.BlockSpec((1, _D), lambda i, g: (0, 0)),
        ],
        out_specs=pl.BlockSpec((_RBS, _D), lambda i, g: (i, 0)),
        out_shape=jax.ShapeDtypeStruct((_NI_BOOST * _RBS, _D), jnp.float32),
        compiler_params=pltpu.CompilerParams(
            dimension_semantics=("parallel", "arbitrary")),
    )(q, k, k, k, v, v, v, masks1[0], masks1[1], masks1[2], km, W_o, bo2)

    n2 = _NI - _NI_BOOST
    out2 = pl.pallas_call(
        _make_attn_kernel(False),
        grid=(n2, _KV),
        in_specs=[
            pl.BlockSpec((_G, _RBS, _HD), lambda i, g: (g, i + _NI_BOOST, 0)),
            kv_part(0, 1), kv_part(1, 1), kv_part(2, 1),
            kv_part(0, 1), kv_part(1, 1), kv_part(2, 1),
            pl.BlockSpec((1, _QR, _RBS), lambda i, g: (0, 0, 0)),
            pl.BlockSpec((1, _QR, _RBS), lambda i, g: (0, 0, 0)),
            pl.BlockSpec((1, _QR, _RBS), lambda i, g: (0, 0, 0)),
            pl.BlockSpec((_G * _HD, _D), lambda i, g: (g, 0)),
            pl.BlockSpec((1, _D), lambda i, g: (0, 0)),
        ],
        out_specs=pl.BlockSpec((_RBS, _D), lambda i, g: (i, 0)),
        out_shape=jax.ShapeDtypeStruct((n2 * _RBS, _D), jnp.float32),
        compiler_params=pltpu.CompilerParams(
            dimension_semantics=("parallel", "arbitrary")),
    )(q, k, k, k, v, v, v, masks2[0], masks2[1], masks2[2], W_o, bo2)

    return jnp.concatenate([out1, out2], axis=0).reshape(_B, _T, _D)


# speedup vs baseline: 1.0276x; 1.0084x over previous
import numpy as np

import jax
import jax.numpy as jnp
from jax.experimental import pallas as pl
from jax.experimental.pallas import tpu as pltpu

_B, _T, _D = 1, 2048, 1024
_H, _KV, _HD = 16, 4, 64
_COMP = 256
_SW = 512
_SBS = 128
_SNB = 8
_EPS = 1e-6
_SCALE = _HD ** -0.5
_NBLK = _T // _SBS
_G = _H // _KV
_KD = _KV * _HD

_TS = 256
_RBS = 256
_NP = 3
_NI = _T // _RBS
_QR = _G * _RBS
_NSUB = _TS // _SBS
_NI_BOOST = 3
_NEG = -1e30


def _proj_kernel(x_ref, wc_ref, bc_ref, wdk_ref, bdk_ref, wdv_ref, bdv_ref,
                 wq_ref, bq_ref, wkn_ref, wqn_ref, cos_ref, sin_ref,
                 sk_ref, sq_ref, sm_ref,
                 q_ref, k_ref, v_ref, km_ref):
    i = pl.program_id(0)
    x = x_ref[...]
    lat = jnp.dot(x, wc_ref[...], preferred_element_type=jnp.float32) + bc_ref[...]
    ka = jnp.dot(lat, wdk_ref[...], preferred_element_type=jnp.float32) + bdk_ref[...]
    v = jnp.dot(lat, wdv_ref[...], preferred_element_type=jnp.float32) + bdv_ref[...]
    qa = jnp.dot(x, wq_ref[...], preferred_element_type=jnp.float32) + bq_ref[...]
    cos = cos_ref[...]
    sin = sin_ref[...]

    def norm_rope_full(a, s_ref, w, n):
        ss = jnp.dot(a * a, s_ref[...], preferred_element_type=jnp.float32)
        inv = jax.lax.rsqrt(ss * (1.0 / _HD) + _EPS)
        t = a * w
        tsw = jnp.concatenate(
            [t[:, g * 32 + 32:g * 32 + 64] if g % 2 == 0
             else t[:, g * 32 - 32:g * 32] for g in range(2 * n)], axis=1)
        ct = jnp.concatenate([cos] * n, axis=1)
        st = jnp.concatenate([sin] * n, axis=1)
        return (t * ct + tsw * st) * inv

    k = norm_rope_full(ka, sk_ref, wkn_ref[...], _KV)
    q = norm_rope_full(qa, sq_ref, wqn_ref[...], _H)
    means = jnp.dot(sm_ref[...], k, preferred_element_type=jnp.float32)
    for h in range(_KV):
        k_ref[h] = k[:, h * _HD:(h + 1) * _HD]
        v_ref[h] = v[:, h * _HD:(h + 1) * _HD]
        km_ref[h, pl.ds(i * _NSUB, _NSUB), :] = means[:, h * _HD:(h + 1) * _HD]
    for h in range(_H):
        q_ref[h] = q[:, h * _HD:(h + 1) * _HD]


def _topk_boost_t(bst):
    rowid = jax.lax.broadcasted_iota(jnp.int32, bst.shape, 0)
    sel = []
    for c in range(_NBLK):
        bc = bst[c:c + 1, :]
        gt = jnp.sum((bst > bc).astype(jnp.float32), axis=0, keepdims=True)
        eq = jnp.sum(((bst == bc) & (rowid < c)).astype(jnp.float32),
                     axis=0, keepdims=True)
        sel.append((gt + eq < float(_SNB)).astype(jnp.float32))
    selt = jnp.concatenate(sel, axis=0)
    return jnp.transpose(bst * selt)


def _make_attn_kernel(with_boost):
    def body(*refs):
        if with_boost:
            (q_ref, k0, k1, k2, v0, v1, v2, m0, m1, m2, km_ref, wo_ref,
             bo_ref, out_ref) = refs
        else:
            (q_ref, k0, k1, k2, v0, v1, v2, m0, m1, m2, wo_ref,
             bo_ref, out_ref) = refs
        g = pl.program_id(1)
        qs = q_ref[...].reshape(_QR, _HD)
        sparts = []
        for kp, mp in zip((k0, k1, k2), (m0, m1, m2)):
            sp = jax.lax.dot_general(qs, kp[0], (((1,), (1,)), ((), ())),
                                     preferred_element_type=jnp.float32)
            sparts.append(sp + mp[0])

        if with_boost:
            bst = jax.lax.dot_general(km_ref[0], qs, (((1,), (1,)), ((), ())),
                                      preferred_element_type=jnp.float32)
            boost = _topk_boost_t(bst)
            s0 = sparts[0]
            sparts[0] = jnp.concatenate(
                [s0[:, :_NBLK] + boost, s0[:, _NBLK:]], axis=1)

        m = jnp.maximum(
            jnp.maximum(jnp.max(sparts[0], axis=1, keepdims=True),
                        jnp.max(sparts[1], axis=1, keepdims=True)),
            jnp.max(sparts[2], axis=1, keepdims=True))
        o = None
        l = None
        for sp, vp in zip(sparts, (v0, v1, v2)):
            pp = jnp.exp(sp - m)
            ls = jnp.sum(pp, axis=1, keepdims=True)
            op = jnp.dot(pp, vp[0], preferred_element_type=jnp.float32)
            o = op if o is None else o + op
            l = ls if l is None else l + ls
        o = o * (1.0 / l)
        o_cat = jnp.concatenate(
            [o[hl * _RBS:(hl + 1) * _RBS] for hl in range(_G)], axis=1)
        proj = jnp.dot(o_cat, wo_ref[...], preferred_element_type=jnp.float32)

        @pl.when(g == 0)
        def _init():
            out_ref[...] = proj + bo_ref[...]

        @pl.when(g > 0)
        def _acc():
            out_ref[...] += proj

    return body


def _band_masks():
    r = np.arange(_QR)[:, None] % _RBS
    c = np.arange(_RBS)[None, :]

    def madd(abs_i, colblock):
        rows = abs_i * _RBS + r
        cols = colblock * _RBS + c
        ok = (cols <= rows) & (cols >= rows - _SW)
        return np.where(ok, 0.0, _NEG).astype(np.float32)

    m1 = [np.stack([madd(i, p) for i in range(_NI_BOOST)])
          for p in range(_NP)]
    m2 = [madd(3, 1 + p)[None] for p in range(_NP)]
    return [jnp.asarray(m) for m in m1], [jnp.asarray(m) for m in m2]


def kernel(x, W_comp, b_comp, W_dk, b_dk, W_dv, b_dv, W_q, b_q, W_o, b_o,
           q_norm_w, k_norm_w):
    x2 = x[0]

    perm_h = np.concatenate([np.arange(0, _HD, 2), np.arange(1, _HD, 2)])
    perm_q = (np.arange(_H)[:, None] * _HD + perm_h[None, :]).reshape(-1)
    perm_k = (np.arange(_KV)[:, None] * _HD + perm_h[None, :]).reshape(-1)
    W_qA = W_q[:, perm_q]
    b_qA = b_q[perm_q][None, :]
    W_dkA = W_dk[:, perm_k]
    b_dkA = b_dk[perm_k][None, :]
    wqn = jnp.tile(q_norm_w[perm_h], (_H,))[None, :] * _SCALE
    wkn = jnp.tile(k_norm_w[perm_h], (_KV,))[None, :]

    pos = np.arange(_T, dtype=np.float32)
    inv_freq = 1.0 / (10000.0 ** (np.arange(0, _HD, 2, dtype=np.float32) / _HD))
    ang = pos[:, None] * inv_freq[None, :]
    cos64 = jnp.asarray(np.concatenate([np.cos(ang), np.cos(ang)], axis=1))
    sin64 = jnp.asarray(np.concatenate([-np.sin(ang), np.sin(ang)], axis=1))

    gid_k = np.arange(_KD) // _HD
    sk = jnp.asarray((gid_k[:, None] == gid_k[None, :]).astype(np.float32))
    gid_q = np.arange(_D) // _HD
    sq = jnp.asarray((gid_q[:, None] == gid_q[None, :]).astype(np.float32))
    sub = np.arange(_TS) // _SBS
    sm = jnp.asarray((np.arange(_NSUB)[:, None] == sub[None, :])
                     .astype(np.float32) / _SBS)

    q, k, v, km = pl.pallas_call(
        _proj_kernel,
        grid=(_T // _TS,),
        in_specs=[
            pl.BlockSpec((_TS, _D), lambda i: (i, 0)),
            pl.BlockSpec((_D, _COMP), lambda i: (0, 0)),
            pl.BlockSpec((1, _COMP), lambda i: (0, 0)),
            pl.BlockSpec((_COMP, _KD), lambda i: (0, 0)),
            pl.BlockSpec((1, _KD), lambda i: (0, 0)),
            pl.BlockSpec((_COMP, _KD), lambda i: (0, 0)),
            pl.BlockSpec((1, _KD), lambda i: (0, 0)),
            pl.BlockSpec((_D, _D), lambda i: (0, 0)),
            pl.BlockSpec((1, _D), lambda i: (0, 0)),
            pl.BlockSpec((1, _KD), lambda i: (0, 0)),
            pl.BlockSpec((1, _D), lambda i: (0, 0)),
            pl.BlockSpec((_TS, _HD), lambda i: (i, 0)),
            pl.BlockSpec((_TS, _HD), lambda i: (i, 0)),
            pl.BlockSpec((_KD, _KD), lambda i: (0, 0)),
            pl.BlockSpec((_D, _D), lambda i: (0, 0)),
            pl.BlockSpec((_NSUB, _TS), lambda i: (0, 0)),
        ],
        out_specs=[
            pl.BlockSpec((_H, _TS, _HD), lambda i: (0, i, 0)),
            pl.BlockSpec((_KV, _TS, _HD), lambda i: (0, i, 0)),
            pl.BlockSpec((_KV, _TS, _HD), lambda i: (0, i, 0)),
            pl.BlockSpec((_KV, _NBLK, _HD), lambda i: (0, 0, 0)),
        ],
        out_shape=[
            jax.ShapeDtypeStruct((_H, _T, _HD), jnp.float32),
            jax.ShapeDtypeStruct((_KV, _T, _HD), jnp.float32),
            jax.ShapeDtypeStruct((_KV, _T, _HD), jnp.float32),
            jax.ShapeDtypeStruct((_KV, _NBLK, _HD), jnp.float32),
        ],
        compiler_params=pltpu.CompilerParams(
            dimension_semantics=("arbitrary",)),
    )(x2, W_comp, b_comp.reshape(1, -1), W_dkA, b_dkA,
      W_dv, b_dv.reshape(1, -1), W_qA, b_qA, wkn, wqn, cos64, sin64,
      sk, sq, sm)

    bo2 = b_o.reshape(1, -1)
    masks1, masks2 = _band_masks()

    def kv_part(p, shift):
        if shift is None:
            return pl.BlockSpec((1, _RBS, _HD), lambda i, g, p=p: (g, p, 0))
        return pl.BlockSpec((1, _RBS, _HD),
                            lambda i, g, p=p, s=shift: (g, i + s + p, 0))

    out1 = pl.pallas_call(
        _make_attn_kernel(True),
        grid=(_NI_BOOST, _KV),
        in_specs=[
            pl.BlockSpec((_G, _RBS, _HD), lambda i, g: (g, i, 0)),
            kv_part(0, None), kv_part(1, None), kv_part(2, None),
            kv_part(0, None), kv_part(1, None), kv_part(2, None),
            pl.BlockSpec((1, _QR, _RBS), lambda i, g: (i, 0, 0)),
            pl.BlockSpec((1, _QR, _RBS), lambda i, g: (i, 0, 0)),
            pl.BlockSpec((1, _QR, _RBS), lambda i, g: (i, 0, 0)),
            pl.BlockSpec((1, _NBLK, _HD), lambda i, g: (g, 0, 0)),
            pl.BlockSpec((_G * _HD, _D), lambda i, g: (g, 0)),
            pl.BlockSpec((1, _D), lambda i, g: (0, 0)),
        ],
        out_specs=pl.BlockSpec((_RBS, _D), lambda i, g: (i, 0)),
        out_shape=jax.ShapeDtypeStruct((_NI_BOOST * _RBS, _D), jnp.float32),
        compiler_params=pltpu.CompilerParams(
            dimension_semantics=("parallel", "arbitrary")),
    )(q, k, k, k, v, v, v, masks1[0], masks1[1], masks1[2], km, W_o, bo2)

    n2 = _NI - _NI_BOOST
    out2 = pl.pallas_call(
        _make_attn_kernel(False),
        grid=(n2, _KV),
        in_specs=[
            pl.BlockSpec((_G, _RBS, _HD), lambda i, g: (g, i + _NI_BOOST, 0)),
            kv_part(0, 1), kv_part(1, 1), kv_part(2, 1),
            kv_part(0, 1), kv_part(1, 1), kv_part(2, 1),
            pl.BlockSpec((1, _QR, _RBS), lambda i, g: (0, 0, 0)),
            pl.BlockSpec((1, _QR, _RBS), lambda i, g: (0, 0, 0)),
            pl.BlockSpec((1, _QR, _RBS), lambda i, g: (0, 0, 0)),
            pl.BlockSpec((_G * _HD, _D), lambda i, g: (g, 0)),
            pl.BlockSpec((1, _D), lambda i, g: (0, 0)),
        ],
        out_specs=pl.BlockSpec((_RBS, _D), lambda i, g: (i, 0)),
        out_shape=jax.ShapeDtypeStruct((n2 * _RBS, _D), jnp.float32),
        compiler_params=pltpu.CompilerParams(
            dimension_semantics=("parallel", "arbitrary")),
    )(q, k, k, k, v, v, v, masks2[0], masks2[1], masks2[2], W_o, bo2)

    return jnp.concatenate([out1, out2], axis=0).reshape(_B, _T, _D)
